# baseline (speedup 1.0000x reference)
import jax
import jax.numpy as jnp
from jax.experimental import pallas as pl
from jax.experimental.pallas import tpu as pltpu

_DD = 1


def _tc_body(x_ref, t_ref, o_ref):
    C = x_ref.shape[1]
    W = x_ref.shape[4]
    x = x_ref[0]
    tgt = t_ref[0]
    best = x[0]
    pred = jnp.zeros_like(tgt)
    for c in range(1, C):
        m = x[c] > best
        best = jnp.where(m, x[c], best)
        pred = jnp.where(m, c, pred)
    one = jnp.ones_like(best)
    zero = jnp.zeros_like(best)
    rows = []
    for c in range(C):
        pc = pred == c
        tc = tgt == c
        for msk in (pc & tc, pc, tc):
            r = jnp.sum(jnp.where(msk, one, zero), axis=(0, 1), keepdims=True)
            rows.append(r.reshape(1, W))
    cnt = jnp.concatenate(rows, axis=0)
    i = pl.program_id(1)

    @pl.when(i == 0)
    def _init():
        o_ref[0] = cnt

    @pl.when(i > 0)
    def _acc():
        o_ref[0] = o_ref[0] + cnt


def kernel(inputs, targets):
    eps = 1e-05
    B, C, D, H, W = inputs.shape
    t = targets.astype(jnp.int32)
    G = D // _DD
    counts = pl.pallas_call(
        _tc_body,
        grid=(B, G),
        in_specs=[
            pl.BlockSpec((1, C, _DD, H, W), lambda b, i: (b, 0, i, 0, 0)),
            pl.BlockSpec((1, _DD, H, W), lambda b, i: (b, i, 0, 0)),
        ],
        out_specs=pl.BlockSpec((1, 3 * C, W), lambda b, i: (b, 0, 0)),
        out_shape=jax.ShapeDtypeStruct((B, 3 * C, W), jnp.float32),
        compiler_params=pltpu.CompilerParams(
            dimension_semantics=("parallel", "arbitrary")),
    )(inputs, t)
    cnt = counts.sum(axis=2).reshape(B, C, 3)
    tp, cp, ct = cnt[..., 0], cnt[..., 1], cnt[..., 2]
    loss = 2.0 * tp / (cp + ct + eps)
    return loss[:, 1:].mean(axis=1)


# speedup vs baseline: 1.1998x; 1.1998x over previous
import functools

import jax
import jax.numpy as jnp
from jax import lax
from jax.experimental import pallas as pl
from jax.experimental.pallas import tpu as pltpu
from jax.experimental.pallas import tpu_sc as plsc

_NC, _NS = 2, 16
_NW = _NC * _NS
_DD = 2
_D_SC = 8
_RB = 24


def _sc_counts(x3, t3, B, C, D, H, W, d0):
    d_sc = D - d0
    tasks_pb = d_sc * (H // _RB)
    wpb = _NW // B
    T = tasks_pb // wpb
    mesh = plsc.VectorSubcoreMesh(core_axis_name="c", subcore_axis_name="s")

    @functools.partial(
        pl.kernel, mesh=mesh,
        out_type=jax.ShapeDtypeStruct((_NW * 3 * 16,), jnp.float32),
        scratch_types=[
            pltpu.VMEM((C, _RB, W), jnp.float32),
            pltpu.VMEM((_RB, W), jnp.int32),
        ] + [pltpu.VMEM((16,), jnp.float32) for _ in range(3)],
        compiler_params=pltpu.CompilerParams(needs_layout_passes=False),
    )
    def k(x_hbm, t_hbm, out_hbm, xbuf, tbuf, acc_tp, acc_cp, acc_ct):
        wid = lax.axis_index("s") * _NC + lax.axis_index("c")
        b = wid // wpb
        local = wid % wpb
        zero16 = jnp.zeros((16,), jnp.float32)
        ones = jnp.ones((16,), jnp.float32)
        acc_tp[...] = zero16
        acc_cp[...] = zero16
        acc_ct[...] = zero16

        def task_body(t, carry):
            pt = local * T + t
            ds = pt // (H // _RB)
            rb = pt % (H // _RB)
            d = d0 + ds
            r0 = rb * _RB
            for c in range(C):
                pltpu.sync_copy(
                    x_hbm.at[(b * C + c) * D + d, pl.ds(r0, _RB)],
                    xbuf.at[c])
            pltpu.sync_copy(t_hbm.at[b * D + d, pl.ds(r0, _RB)], tbuf)

            def row_body(r, carry2):
                def vec_body(j, carry3):
                    s = pl.ds(j * 16, 16)
                    best = xbuf[0, r, s]
                    pred = jnp.zeros((16,), jnp.int32)
                    for c in range(1, C):
                        xc = xbuf[c, r, s]
                        m = xc > best
                        best = jnp.where(m, xc, best)
                        pred = jnp.where(m, c, pred)
                    tg = tbuf[r, s]
                    eq = pred == tg
                    plsc.addupdate_scatter(acc_tp, [tg], ones, mask=eq)
                    plsc.addupdate_scatter(acc_cp, [pred], ones)
                    plsc.addupdate_scatter(acc_ct, [tg], ones)
                    return carry3

                return lax.fori_loop(0, W // 16, vec_body, carry2)

            return lax.fori_loop(0, _RB, row_body, carry)

        lax.fori_loop(0, T, task_body, 0)
        for r, a in enumerate((acc_tp, acc_cp, acc_ct)):
            pltpu.sync_copy(a, out_hbm.at[pl.ds((wid * 3 + r) * 16, 16)])

    return k(x3, t3)


def _tc_body(x_ref, t_ref, o_ref):
    C = x_ref.shape[1]
    W = x_ref.shape[4]
    x = x_ref[0]
    tgt = t_ref[0]
    best = x[0]
    pred = jnp.zeros_like(tgt)
    for c in range(1, C):
        m = x[c] > best
        best = jnp.where(m, x[c], best)
        pred = jnp.where(m, c, pred)
    one = jnp.ones_like(best)
    zero = jnp.zeros_like(best)
    rows = []
    for c in range(C):
        pc = pred == c
        tc = tgt == c
        for msk in (pc & tc, pc, tc):
            r = jnp.sum(jnp.where(msk, one, zero), axis=(0, 1), keepdims=True)
            rows.append(r.reshape(1, W))
    cnt = jnp.concatenate(rows, axis=0)
    i = pl.program_id(1)

    @pl.when(i == 0)
    def _init():
        o_ref[0] = cnt

    @pl.when(i > 0)
    def _acc():
        o_ref[0] = o_ref[0] + cnt


def kernel(inputs, targets):
    eps = 1e-05
    B, C, D, H, W = inputs.shape
    t = targets.astype(jnp.int32)
    d0 = D - _D_SC

    x3 = inputs.reshape(B * C * D, H, W)
    t3 = t.reshape(B * D, H, W)
    sc_parts = _sc_counts(x3, t3, B, C, D, H, W, d0)

    G = d0 // _DD
    counts = pl.pallas_call(
        _tc_body,
        grid=(B, G),
        in_specs=[
            pl.BlockSpec((1, C, _DD, H, W), lambda b, i: (b, 0, i, 0, 0)),
            pl.BlockSpec((1, _DD, H, W), lambda b, i: (b, i, 0, 0)),
        ],
        out_specs=pl.BlockSpec((1, 3 * C, W), lambda b, i: (b, 0, 0)),
        out_shape=jax.ShapeDtypeStruct((B, 3 * C, W), jnp.float32),
        compiler_params=pltpu.CompilerParams(
            dimension_semantics=("parallel", "arbitrary")),
    )(inputs, t)
    tc_cnt = counts.sum(axis=2).reshape(B, C, 3)

    wpb = _NW // B
    sc_cnt = (sc_parts.reshape(B, wpb, 3, 16).sum(axis=1))[:, :, :C]

    tp = tc_cnt[..., 0] + sc_cnt[:, 0]
    cp = tc_cnt[..., 1] + sc_cnt[:, 1]
    ct = tc_cnt[..., 2] + sc_cnt[:, 2]
    loss = 2.0 * tp / (cp + ct + eps)
    return loss[:, 1:].mean(axis=1)


# speedup vs baseline: 1.2648x; 1.0542x over previous
import functools

import jax
import jax.numpy as jnp
from jax import lax
from jax.experimental import pallas as pl
from jax.experimental.pallas import tpu as pltpu
from jax.experimental.pallas import tpu_sc as plsc

_NC, _NS = 2, 16
_NW = _NC * _NS
_DD = 2
_D_SC = 12
_RB = 24


def _sc_counts(x3, t3, B, C, D, H, W, d0):
    d_sc = D - d0
    tasks_pb = d_sc * (H // _RB)
    wpb = _NW // B
    T = tasks_pb // wpb
    mesh = plsc.VectorSubcoreMesh(core_axis_name="c", subcore_axis_name="s")

    @functools.partial(
        pl.kernel, mesh=mesh,
        out_type=jax.ShapeDtypeStruct((_NW * 3 * 16,), jnp.float32),
        scratch_types=[
            pltpu.VMEM((C, _RB, W), jnp.float32),
            pltpu.VMEM((_RB, W), jnp.int32),
        ] + [pltpu.VMEM((16,), jnp.float32) for _ in range(3)],
        compiler_params=pltpu.CompilerParams(needs_layout_passes=False),
    )
    def k(x_hbm, t_hbm, out_hbm, xbuf, tbuf, acc_tp, acc_cp, acc_ct):
        wid = lax.axis_index("s") * _NC + lax.axis_index("c")
        b = wid // wpb
        local = wid % wpb
        zero16 = jnp.zeros((16,), jnp.float32)
        ones = jnp.ones((16,), jnp.float32)
        acc_tp[...] = zero16
        acc_cp[...] = zero16
        acc_ct[...] = zero16

        def task_body(t, carry):
            pt = local * T + t
            ds = pt // (H // _RB)
            rb = pt % (H // _RB)
            d = d0 + ds
            r0 = rb * _RB
            for c in range(C):
                pltpu.sync_copy(
                    x_hbm.at[(b * C + c) * D + d, pl.ds(r0, _RB)],
                    xbuf.at[c])
            pltpu.sync_copy(t_hbm.at[b * D + d, pl.ds(r0, _RB)], tbuf)

            def row_body(r, carry2):
                def vec_body(j, carry3):
                    s = pl.ds(j * 16, 16)
                    best = xbuf[0, r, s]
                    pred = jnp.zeros((16,), jnp.int32)
                    for c in range(1, C):
                        xc = xbuf[c, r, s]
                        m = xc > best
                        best = jnp.where(m, xc, best)
                        pred = jnp.where(m, c, pred)
                    tg = tbuf[r, s]
                    eq = pred == tg
                    plsc.addupdate_scatter(acc_tp, [tg], ones, mask=eq)
                    plsc.addupdate_scatter(acc_cp, [pred], ones)
                    plsc.addupdate_scatter(acc_ct, [tg], ones)
                    return carry3

                return lax.fori_loop(0, W // 16, vec_body, carry2)

            return lax.fori_loop(0, _RB, row_body, carry)

        lax.fori_loop(0, T, task_body, 0)
        for r, a in enumerate((acc_tp, acc_cp, acc_ct)):
            pltpu.sync_copy(a, out_hbm.at[pl.ds((wid * 3 + r) * 16, 16)])

    return k(x3, t3)


def _tc_body(x_ref, t_ref, o_ref):
    C = x_ref.shape[1]
    W = x_ref.shape[4]
    x = x_ref[0]
    tgt = t_ref[0]
    best = x[0]
    pred = jnp.zeros_like(tgt)
    for c in range(1, C):
        m = x[c] > best
        best = jnp.where(m, x[c], best)
        pred = jnp.where(m, c, pred)
    one = jnp.ones_like(best)
    zero = jnp.zeros_like(best)
    rows = []
    for c in range(C):
        pc = pred == c
        tc = tgt == c
        for msk in (pc & tc, pc, tc):
            r = jnp.sum(jnp.where(msk, one, zero), axis=(0, 1), keepdims=True)
            rows.append(r.reshape(1, W))
    cnt = jnp.concatenate(rows, axis=0)
    i = pl.program_id(1)

    @pl.when(i == 0)
    def _init():
        o_ref[0] = cnt

    @pl.when(i > 0)
    def _acc():
        o_ref[0] = o_ref[0] + cnt


def kernel(inputs, targets):
    eps = 1e-05
    B, C, D, H, W = inputs.shape
    t = targets.astype(jnp.int32)
    d0 = D - _D_SC

    x3 = inputs.reshape(B * C * D, H, W)
    t3 = t.reshape(B * D, H, W)
    sc_parts = _sc_counts(x3, t3, B, C, D, H, W, d0)

    G = d0 // _DD
    counts = pl.pallas_call(
        _tc_body,
        grid=(B, G),
        in_specs=[
            pl.BlockSpec((1, C, _DD, H, W), lambda b, i: (b, 0, i, 0, 0)),
            pl.BlockSpec((1, _DD, H, W), lambda b, i: (b, i, 0, 0)),
        ],
        out_specs=pl.BlockSpec((1, 3 * C, W), lambda b, i: (b, 0, 0)),
        out_shape=jax.ShapeDtypeStruct((B, 3 * C, W), jnp.float32),
        compiler_params=pltpu.CompilerParams(
            dimension_semantics=("parallel", "arbitrary")),
    )(inputs, t)
    tc_cnt = counts.sum(axis=2).reshape(B, C, 3)

    wpb = _NW // B
    sc_cnt = (sc_parts.reshape(B, wpb, 3, 16).sum(axis=1))[:, :, :C]

    tp = tc_cnt[..., 0] + sc_cnt[:, 0]
    cp = tc_cnt[..., 1] + sc_cnt[:, 1]
    ct = tc_cnt[..., 2] + sc_cnt[:, 2]
    loss = 2.0 * tp / (cp + ct + eps)
    return loss[:, 1:].mean(axis=1)
